# baseline (speedup 1.0000x reference)
import functools
import math

import jax
import jax.numpy as jnp
from jax.experimental import pallas as pl
from jax.experimental.pallas import tpu as pltpu

_TOP_K = 64
_BLK = 512


def _score_kernel(nb, n_seeds, x_ref, wqt_ref, seeds_ref, klog_ref, out_ref, q_ref):
    j = pl.program_id(0)

    @pl.when(j == 0)
    def _():
        q_ref[...] = jnp.dot(x_ref[...], wqt_ref[...])

    s = jax.lax.dot_general(
        q_ref[...], seeds_ref[...],
        (((1,), (1,)), ((), ())),
    ) * (1.0 / math.sqrt(x_ref.shape[1])) + klog_ref[0]

    rem = n_seeds - (nb - 1) * _BLK

    @pl.when(j < nb - 1)
    def _():
        out_ref[...] = s

    @pl.when(j == nb - 1)
    def _():
        col = jax.lax.broadcasted_iota(jnp.int32, s.shape, 1)
        out_ref[...] = jnp.where(col < rem, s, -jnp.inf)


def _scores(x, seeds, wqt, klog):
    b, d = x.shape
    n = seeds.shape[0]
    nb = (n + _BLK - 1) // _BLK
    klog3 = jnp.pad(klog, (0, nb * _BLK - n)).reshape(nb, 1, _BLK)
    return pl.pallas_call(
        functools.partial(_score_kernel, nb, n),
        grid=(nb,),
        in_specs=[
            pl.BlockSpec((b, d), lambda j: (0, 0)),
            pl.BlockSpec((d, d), lambda j: (0, 0)),
            pl.BlockSpec((_BLK, d), lambda j: (j, 0)),
            pl.BlockSpec((1, 1, _BLK), lambda j: (j, 0, 0)),
        ],
        out_specs=pl.BlockSpec((b, _BLK), lambda j: (0, j)),
        out_shape=jax.ShapeDtypeStruct((b, nb * _BLK), jnp.float32),
        scratch_shapes=[pltpu.VMEM((b, d), jnp.float32)],
    )(x, wqt, seeds, klog3)


def kernel(x, seeds, karma, W_q):
    karma_log = jnp.maximum(jnp.log(jax.nn.softmax(karma, axis=-1)), -10.0)
    scores = _scores(x, seeds, W_q.T, karma_log)
    top_scores, top_idx = jax.lax.top_k(scores, _TOP_K)
    attn = jax.nn.softmax(top_scores, axis=-1)
    seeds_k = jnp.take(seeds, top_idx, axis=0)
    field = jnp.sum(attn[..., None] * seeds_k, axis=1)
    return (field, attn)


# speedup vs baseline: 6.3376x; 6.3376x over previous
import functools
import math

import jax
import jax.numpy as jnp
from jax.experimental import pallas as pl
from jax.experimental.pallas import tpu as pltpu

_TOP_K = 64
_BLK = 512
_G = 32
_MPB = _BLK // _G


def _score_kernel(nb, n_seeds, x_ref, wqt_ref, seeds_ref, klog_ref,
                  out_ref, m_ref, q_ref):
    j = pl.program_id(0)

    @pl.when(j == 0)
    def _():
        q_ref[...] = jnp.dot(x_ref[...], wqt_ref[...])

    s = jax.lax.dot_general(
        q_ref[...], seeds_ref[...],
        (((1,), (1,)), ((), ())),
    ) * (1.0 / math.sqrt(x_ref.shape[1])) + klog_ref[0]

    rem = n_seeds - (nb - 1) * _BLK

    @pl.when(j == nb - 1)
    def _():
        col = jax.lax.broadcasted_iota(jnp.int32, s.shape, 1)
        out_ref[...] = jnp.where(col < rem, s, -jnp.inf)
        m_ref[0] = jnp.max(out_ref[...].reshape(s.shape[0], _MPB, _G), axis=2)

    @pl.when(j < nb - 1)
    def _():
        out_ref[...] = s
        m_ref[0] = jnp.max(s.reshape(s.shape[0], _MPB, _G), axis=2)


def _scores(x, seeds, wqt, klog):
    b, d = x.shape
    n = seeds.shape[0]
    nb = (n + _BLK - 1) // _BLK
    klog3 = jnp.pad(klog, (0, nb * _BLK - n)).reshape(nb, 1, _BLK)
    return pl.pallas_call(
        functools.partial(_score_kernel, nb, n),
        grid=(nb,),
        in_specs=[
            pl.BlockSpec((b, d), lambda j: (0, 0)),
            pl.BlockSpec((d, d), lambda j: (0, 0)),
            pl.BlockSpec((_BLK, d), lambda j: (j, 0)),
            pl.BlockSpec((1, 1, _BLK), lambda j: (j, 0, 0)),
        ],
        out_specs=[
            pl.BlockSpec((b, _BLK), lambda j: (0, j)),
            pl.BlockSpec((1, b, _MPB), lambda j: (j, 0, 0)),
        ],
        out_shape=[
            jax.ShapeDtypeStruct((b, nb * _BLK), jnp.float32),
            jax.ShapeDtypeStruct((nb, b, _MPB), jnp.float32),
        ],
        scratch_shapes=[pltpu.VMEM((b, d), jnp.float32)],
    )(x, wqt, seeds, klog3)


def kernel(x, seeds, karma, W_q):
    b = x.shape[0]
    karma_log = jnp.maximum(jnp.log(jax.nn.softmax(karma, axis=-1)), -10.0)
    scores, blkmax3 = _scores(x, seeds, W_q.T, karma_log)
    blkmax = jnp.moveaxis(blkmax3, 0, 1).reshape(b, -1)
    _, top_blk = jax.lax.top_k(blkmax, _TOP_K)
    cand_idx = (top_blk[..., None] * _G
                + jnp.arange(_G, dtype=top_blk.dtype)).reshape(b, _TOP_K * _G)
    cand = jnp.take_along_axis(scores, cand_idx, axis=1)
    top_scores, ci = jax.lax.top_k(cand, _TOP_K)
    top_idx = jnp.take_along_axis(cand_idx, ci, axis=1)
    attn = jax.nn.softmax(top_scores, axis=-1)
    seeds_k = jnp.take(seeds, top_idx, axis=0)
    field = jnp.sum(attn[..., None] * seeds_k, axis=1)
    return (field, attn)


# speedup vs baseline: 8.8842x; 1.4018x over previous
import functools
import math

import jax
import jax.numpy as jnp
from jax import lax
from jax.experimental import pallas as pl
from jax.experimental.pallas import tpu as pltpu
from jax.experimental.pallas import tpu_sc as plsc

_TOP_K = 64
_BLK = 512
_G = 32
_MPB = _BLK // _G


def _score_kernel(nb, n_seeds, x_ref, wqt_ref, seeds_ref, klog_ref,
                  out_ref, m_ref, q_ref):
    j = pl.program_id(0)

    @pl.when(j == 0)
    def _():
        q_ref[...] = jnp.dot(x_ref[...], wqt_ref[...])

    s = jax.lax.dot_general(
        q_ref[...], seeds_ref[...],
        (((1,), (1,)), ((), ())),
    ) * (1.0 / math.sqrt(x_ref.shape[1])) + klog_ref[0]

    rem = n_seeds - (nb - 1) * _BLK

    @pl.when(j == nb - 1)
    def _():
        col = jax.lax.broadcasted_iota(jnp.int32, s.shape, 1)
        out_ref[...] = jnp.where(col < rem, s, -jnp.inf)
        m_ref[0] = jnp.max(out_ref[...].reshape(s.shape[0], _MPB, _G), axis=2)

    @pl.when(j < nb - 1)
    def _():
        out_ref[...] = s
        m_ref[0] = jnp.max(s.reshape(s.shape[0], _MPB, _G), axis=2)


def _scores(x, seeds, wqt, klog):
    b, d = x.shape
    n = seeds.shape[0]
    nb = (n + _BLK - 1) // _BLK
    klog3 = jnp.pad(klog, (0, nb * _BLK - n)).reshape(nb, 1, _BLK)
    return pl.pallas_call(
        functools.partial(_score_kernel, nb, n),
        grid=(nb,),
        in_specs=[
            pl.BlockSpec((b, d), lambda j: (0, 0)),
            pl.BlockSpec((d, d), lambda j: (0, 0)),
            pl.BlockSpec((_BLK, d), lambda j: (j, 0)),
            pl.BlockSpec((1, 1, _BLK), lambda j: (j, 0, 0)),
        ],
        out_specs=[
            pl.BlockSpec((b, _BLK), lambda j: (0, j)),
            pl.BlockSpec((1, b, _MPB), lambda j: (j, 0, 0)),
        ],
        out_shape=[
            jax.ShapeDtypeStruct((b, nb * _BLK), jnp.float32),
            jax.ShapeDtypeStruct((nb, b, _MPB), jnp.float32),
        ],
        scratch_shapes=[pltpu.VMEM((b, d), jnp.float32)],
    )(x, wqt, seeds, klog3)


def _sc_select_combine(s_flat, m_t, seeds, b, d):
    nblk = m_t.shape[1]
    l1n = nblk // 16
    ngrp = b // 16
    gpw = ngrp // 32
    dc = d // 16
    mesh = plsc.VectorSubcoreMesh(core_axis_name="c", subcore_axis_name="s")

    @functools.partial(
        pl.kernel, mesh=mesh,
        compiler_params=pltpu.CompilerParams(use_tc_tiling_on_sc=False,
                                             needs_layout_passes=False),
        out_type=[jax.ShapeDtypeStruct((b, d), jnp.float32),
                  jax.ShapeDtypeStruct((b, _TOP_K), jnp.float32)],
        scratch_types=[
            pltpu.VMEM((nblk, 16), jnp.float32),
            pltpu.VMEM((l1n, 16), jnp.float32),
            pltpu.VMEM((_TOP_K, 16), jnp.int32),
            pltpu.VMEM((_TOP_K, 16), jnp.float32),
            pltpu.VMEM((_TOP_K * 16,), jnp.int32),
            pltpu.VMEM((_TOP_K * 16, _G), jnp.float32),
            pltpu.VMEM((_TOP_K, 16), jnp.float32),
            pltpu.VMEM((_TOP_K * 16,), jnp.int32),
            pltpu.VMEM((16, _TOP_K), jnp.float32),
            pltpu.VMEM((_TOP_K // 2, d), jnp.float32),
            pltpu.VMEM((16, d), jnp.float32),
            pltpu.SemaphoreType.DMA,
        ],
    )
    def body(s_ref, mt_ref, seeds_ref, field_ref, attn_ref,
             mt_buf, l1, bw_id, bw_val, cidx, cand, tv, sidx, aout, srows,
             fbuf, sem):
        wid = lax.axis_index("s") * 2 + lax.axis_index("c")
        lanes = lax.iota(jnp.int32, 16)
        ninf = jnp.full((16,), -jnp.inf, jnp.float32)
        zeros_i = jnp.zeros((16,), jnp.int32)

        def do_group(g, _):
            q0 = g * 16
            pltpu.sync_copy(mt_ref.at[g], mt_buf)

            def l1_step(k, _):
                def mstep(t, acc):
                    return jnp.maximum(acc, mt_buf[k * 16 + t])
                l1[k] = lax.fori_loop(1, 16, mstep, mt_buf[k * 16])
                return 0
            lax.fori_loop(0, l1n, l1_step, 0)

            def ext_step(e, _):
                def chain(k, c):
                    m, kb = c
                    v = l1[k]
                    upd = v > m
                    return (jnp.where(upd, v, m),
                            jnp.where(upd, jnp.full((16,), k, jnp.int32), kb))
                _, kb = lax.fori_loop(0, l1n, chain, (ninf, zeros_i))

                def within(t, c):
                    m2, jb = c
                    idx = kb * 16 + t
                    v = plsc.load_gather(mt_buf, [idx, lanes])
                    upd = v > m2
                    return (jnp.where(upd, v, m2), jnp.where(upd, idx, jb))
                m2, jb = lax.fori_loop(0, 16, within, (ninf, zeros_i))

                bw_id[e] = jb
                bw_val[e] = m2
                plsc.store_scatter(mt_buf, [jb, lanes], ninf)

                def remax(t, acc):
                    return jnp.maximum(
                        acc, plsc.load_gather(mt_buf, [kb * 16 + t, lanes]))
                plsc.store_scatter(
                    l1, [kb, lanes],
                    lax.fori_loop(1, 16, remax,
                                  plsc.load_gather(mt_buf, [kb * 16, lanes])))
                return 0
            lax.fori_loop(0, _TOP_K, ext_step, 0)

            def cidx_step(e, _):
                rid = (q0 + lanes) * nblk + bw_id[e]
                plsc.store_scatter(cidx, [e * 16 + lanes], rid)
                return 0
            lax.fori_loop(0, _TOP_K, cidx_step, 0)
            copies = [
                pltpu.async_copy(s_ref.at[cidx.at[pl.ds(i * 128, 128)]],
                                 cand.at[pl.ds(i * 128, 128)], sem)
                for i in range(_TOP_K * 16 // 128)
            ]
            for cp in copies:
                cp.wait()

            def t_step(e, _):
                def chain(r, c):
                    m, rb = c
                    v = bw_val[r]
                    upd = v > m
                    return (jnp.where(upd, v, m),
                            jnp.where(upd, jnp.full((16,), r, jnp.int32), rb))
                _, rb = lax.fori_loop(0, _TOP_K, chain, (ninf, zeros_i))
                crow = rb * 16 + lanes

                def slots(t, c):
                    mm, m2, sb = c
                    v = plsc.load_gather(cand,
                                         [crow, jnp.full((16,), t, jnp.int32)])
                    upd = v > mm
                    m2n = jnp.where(upd, mm, jnp.maximum(m2, v))
                    mmn = jnp.where(upd, v, mm)
                    sbn = jnp.where(upd, jnp.full((16,), t, jnp.int32), sb)
                    return mmn, m2n, sbn
                mm, m2, sb = lax.fori_loop(0, _G, slots, (ninf, ninf, zeros_i))

                tv[e] = mm
                bid = plsc.load_gather(bw_id, [rb, lanes])
                plsc.store_scatter(sidx, [lanes * _TOP_K + e], bid * _G + sb)
                plsc.store_scatter(cand, [crow, sb], ninf)
                plsc.store_scatter(bw_val, [rb, lanes], m2)
                return 0
            lax.fori_loop(0, _TOP_K, t_step, 0)

            def mx_chain(e, m):
                return jnp.maximum(m, tv[e])
            mx = lax.fori_loop(0, _TOP_K, mx_chain, ninf)

            def exp_step(e, ssum):
                ex = jnp.exp(tv[e] - mx)
                tv[e] = ex
                return ssum + ex
            ssum = lax.fori_loop(0, _TOP_K, exp_step,
                                 jnp.zeros((16,), jnp.float32))

            def norm_step(e, _):
                a = tv[e] / ssum
                plsc.store_scatter(
                    aout, [lanes, jnp.full((16,), e, jnp.int32)], a)
                return 0
            lax.fori_loop(0, _TOP_K, norm_step, 0)
            pltpu.sync_copy(aout, attn_ref.at[pl.ds(q0, 16), :])

            def q_step(l, _):
                accs = tuple(jnp.zeros((16,), jnp.float32) for _ in range(dc))
                for half in range(2):
                    pltpu.async_copy(
                        seeds_ref.at[sidx.at[pl.ds(l * _TOP_K + half * 32, 32)]],
                        srows, sem).wait()

                    def e_step(e, accs, half=half):
                        w = plsc.load_gather(
                            aout, [jnp.full((16,), l, jnp.int32),
                                   jnp.full((16,), half * 32, jnp.int32) + e])
                        return tuple(accs[c] + w * srows[e, pl.ds(c * 16, 16)]
                                     for c in range(dc))
                    accs = lax.fori_loop(0, 32, e_step, accs)
                for c in range(dc):
                    plsc.store_scatter(
                        fbuf, [jnp.full((16,), l, jnp.int32),
                               c * 16 + lanes], accs[c])
                return 0
            lax.fori_loop(0, 16, q_step, 0)
            pltpu.sync_copy(fbuf, field_ref.at[pl.ds(q0, 16), :])
            return 0

        lax.fori_loop(wid * gpw, (wid + 1) * gpw, do_group, 0)

    return body(s_flat, m_t, seeds)


def kernel(x, seeds, karma, W_q):
    b, d = x.shape
    karma_log = jnp.maximum(jnp.log(jax.nn.softmax(karma, axis=-1)), -10.0)
    scores, blkmax3 = _scores(x, seeds, W_q.T, karma_log)
    nb = blkmax3.shape[0]
    m_t = (blkmax3.reshape(nb, b // 16, 16, _MPB)
           .transpose(1, 0, 3, 2).reshape(b // 16, nb * _MPB, 16))
    field, attn = _sc_select_combine(
        scores.reshape(-1, _G), m_t, seeds, b, d)
    return (field, attn)
